# baseline (speedup 1.0000x reference)
import functools

import jax
import jax.numpy as jnp
from jax import lax
from jax.experimental import pallas as pl
from jax.experimental.pallas import tpu as pltpu
from jax.experimental.pallas import tpu_sc as plsc

_NROWS = 8
_HIDDEN = 1024
_B = 4 * 4096
_NC, _NS = 2, 16
_NW = _NC * _NS
_BPW = _B // _NW
_CH = 32
_NCHUNK = _BPW // _CH

_mesh = plsc.VectorSubcoreMesh(core_axis_name="c", subcore_axis_name="s")


@functools.partial(
    pl.kernel,
    mesh=_mesh,
    out_type=jax.ShapeDtypeStruct((_B, _HIDDEN), jnp.float32),
    scratch_types=[
        pltpu.VMEM((_NCHUNK, _CH), jnp.int32),
        pltpu.VMEM((_CH, _HIDDEN), jnp.float32),
        pltpu.VMEM((_CH, _HIDDEN), jnp.float32),
        pltpu.SemaphoreType.DMA,
        pltpu.SemaphoreType.DMA,
        pltpu.SemaphoreType.DMA,
        pltpu.SemaphoreType.DMA,
    ],
)
def _sc_gather(idx_hbm, table_hbm, out_hbm, idx_v, buf0, buf1, g0, g1, s0, s1):
    wid = lax.axis_index("s") * _NC + lax.axis_index("c")
    base = wid * _BPW
    pltpu.sync_copy(idx_hbm.at[pl.ds(wid * _NCHUNK, _NCHUNK)], idx_v)
    bufs = (buf0, buf1)
    gsem = (g0, g1)
    ssem = (s0, s1)
    gat = [None, None]
    sto = [None, None]
    gat[0] = pltpu.async_copy(table_hbm.at[idx_v.at[0]], buf0, g0)
    for j in range(_NCHUNK):
        cur = j & 1
        nxt = 1 - cur
        if j + 1 < _NCHUNK:
            if sto[nxt] is not None:
                sto[nxt].wait()
            gat[nxt] = pltpu.async_copy(
                table_hbm.at[idx_v.at[j + 1]], bufs[nxt], gsem[nxt])
        gat[cur].wait()
        sto[cur] = pltpu.async_copy(
            bufs[cur], out_hbm.at[pl.ds(base + j * _CH, _CH)], ssem[cur])
    sto[0].wait()
    sto[1].wait()


def kernel(ids, indicator_embs):
    stride = _NROWS + 1
    table_rep = jnp.pad(
        jnp.tile(indicator_embs, (_NW, 1)).reshape(_NW, _NROWS, _HIDDEN),
        ((0, 0), (0, 1), (0, 0))).reshape(_NW * stride, _HIDDEN)
    ids_2d = ids.reshape(_NW * _NCHUNK, _CH).astype(jnp.int32)
    bias = stride * (jnp.arange(_NW * _NCHUNK, dtype=jnp.int32) // _NCHUNK)
    ids_2d = ids_2d + bias[:, None]
    out = _sc_gather(ids_2d, table_rep)
    return out.reshape(ids.shape + (_HIDDEN,))


# speedup vs baseline: 1.0463x; 1.0463x over previous
import functools

import jax
import jax.numpy as jnp
from jax import lax
from jax.experimental import pallas as pl
from jax.experimental.pallas import tpu as pltpu
from jax.experimental.pallas import tpu_sc as plsc

_NROWS = 8
_HIDDEN = 1024
_B = 4 * 4096
_NC, _NS = 2, 16
_NW = _NC * _NS
_BPW = _B // _NW
_CH = 32
_NCHUNK = _BPW // _CH
_NBUF = 3

_mesh = plsc.VectorSubcoreMesh(core_axis_name="c", subcore_axis_name="s")


@functools.partial(
    pl.kernel,
    mesh=_mesh,
    out_type=jax.ShapeDtypeStruct((_B, _HIDDEN), jnp.float32),
    scratch_types=(
        [pltpu.VMEM((_NCHUNK, _CH), jnp.int32)]
        + [pltpu.VMEM((_CH, _HIDDEN), jnp.float32) for _ in range(_NBUF)]
        + [pltpu.SemaphoreType.DMA for _ in range(2 * _NBUF)]
    ),
)
def _sc_gather(idx_hbm, table_hbm, out_hbm, idx_v, *bufs_and_sems):
    bufs = bufs_and_sems[:_NBUF]
    gsem = bufs_and_sems[_NBUF:2 * _NBUF]
    ssem = bufs_and_sems[2 * _NBUF:]
    wid = lax.axis_index("s") * _NC + lax.axis_index("c")
    base = wid * _BPW
    pltpu.sync_copy(idx_hbm.at[pl.ds(wid * _NCHUNK, _NCHUNK)], idx_v)
    gat = [None] * _NBUF
    sto = [None] * _NBUF
    for t in range(_NBUF - 1):
        gat[t] = pltpu.async_copy(table_hbm.at[idx_v.at[t]], bufs[t], gsem[t])
    for j in range(_NCHUNK):
        b = j % _NBUF
        fj = j + _NBUF - 1
        fb = fj % _NBUF
        if fj < _NCHUNK:
            if sto[fb] is not None:
                sto[fb].wait()
            gat[fb] = pltpu.async_copy(
                table_hbm.at[idx_v.at[fj]], bufs[fb], gsem[fb])
        gat[b].wait()
        sto[b] = pltpu.async_copy(
            bufs[b], out_hbm.at[pl.ds(base + j * _CH, _CH)], ssem[b])
    for s in sto:
        if s is not None:
            s.wait()


def kernel(ids, indicator_embs):
    table_rep = jnp.tile(indicator_embs, (_NW, 1))
    ids_2d = ids.reshape(_NW * _NCHUNK, _CH).astype(jnp.int32)
    bias = _NROWS * (jnp.arange(_NW * _NCHUNK, dtype=jnp.int32) // _NCHUNK)
    ids_2d = ids_2d + bias[:, None]
    out = _sc_gather(ids_2d, table_rep)
    return out.reshape(ids.shape + (_HIDDEN,))


# speedup vs baseline: 1.3656x; 1.3052x over previous
import functools

import jax
import jax.numpy as jnp
from jax import lax
from jax.experimental import pallas as pl
from jax.experimental.pallas import tpu as pltpu
from jax.experimental.pallas import tpu_sc as plsc

_NROWS = 8
_HIDDEN = 1024
_B = 4 * 4096
_NC, _NS = 2, 16
_NW = _NC * _NS
_BPW = _B // _NW
_CH = 32
_NCHUNK = _BPW // _CH
_NBUF = 2
_NGROUP = _NCHUNK // _NBUF

_mesh = plsc.VectorSubcoreMesh(core_axis_name="c", subcore_axis_name="s")


@functools.partial(
    pl.kernel,
    mesh=_mesh,
    out_type=jax.ShapeDtypeStruct((_B, _HIDDEN), jnp.float32),
    scratch_types=(
        [pltpu.VMEM((_BPW,), jnp.int32),
         pltpu.VMEM_SHARED((_NROWS, _HIDDEN), jnp.float32)]
        + [pltpu.VMEM((_CH, _HIDDEN), jnp.float32) for _ in range(_NBUF)]
        + [pltpu.SemaphoreType.DMA for _ in range(2 * _NBUF)]
    ),
)
def _sc_gather(idx_hbm, table_hbm, out_hbm, idx_v, table_sh, *bufs_and_sems):
    bufs = bufs_and_sems[:_NBUF]
    gsem = bufs_and_sems[_NBUF:2 * _NBUF]
    ssem = bufs_and_sems[2 * _NBUF:]
    wid = lax.axis_index("s") * _NC + lax.axis_index("c")
    base = wid * _BPW

    @pl.when(lax.axis_index("s") == 0)
    def _():
        pltpu.sync_copy(table_hbm, table_sh)

    pltpu.sync_copy(idx_hbm.at[pl.ds(base, _BPW)], idx_v)
    plsc.subcore_barrier()

    def fill(j, b):
        for h in range(_CH // 16):
            vec = idx_v[pl.ds(j * _CH + h * 16, 16)]
            for i in range(16):
                pltpu.async_copy(
                    table_sh.at[vec[i]], bufs[b].at[h * 16 + i], gsem[b])

    def drain_fill(b):
        pltpu.make_async_copy(
            out_hbm.at[pl.ds(base, _CH)], bufs[b], gsem[b]).wait()

    def store(j, b):
        pltpu.async_copy(
            bufs[b], out_hbm.at[pl.ds(base + j * _CH, _CH)], ssem[b])

    def wait_store(b):
        pltpu.make_async_copy(
            bufs[b], out_hbm.at[pl.ds(base, _CH)], ssem[b]).wait()

    def group(g, carry):
        @pl.when(g > 0)
        def _():
            for b in range(_NBUF):
                wait_store(b)
        for b in range(_NBUF):
            fill(g * _NBUF + b, b)
        for b in range(_NBUF):
            drain_fill(b)
            store(g * _NBUF + b, b)
        return carry

    lax.fori_loop(0, _NGROUP, group, 0)
    for b in range(_NBUF):
        wait_store(b)


def kernel(ids, indicator_embs):
    ids_flat = ids.reshape(_B).astype(jnp.int32)
    out = _sc_gather(ids_flat, indicator_embs)
    return out.reshape(ids.shape + (_HIDDEN,))


# speedup vs baseline: 1.5627x; 1.1444x over previous
import functools

import jax
import jax.numpy as jnp
from jax import lax
from jax.experimental import pallas as pl
from jax.experimental.pallas import tpu as pltpu
from jax.experimental.pallas import tpu_sc as plsc

_NROWS = 8
_HIDDEN = 1024
_B = 4 * 4096
_NC, _NS = 2, 16
_NW = _NC * _NS
_BPW = _B // _NW
_CH = 32
_NCHUNK = _BPW // _CH

_mesh = plsc.VectorSubcoreMesh(core_axis_name="c", subcore_axis_name="s")


@functools.partial(
    pl.kernel,
    mesh=_mesh,
    out_type=jax.ShapeDtypeStruct((_B, _HIDDEN), jnp.float32),
    scratch_types=[
        pltpu.VMEM((_BPW,), jnp.int32),
        pltpu.VMEM_SHARED((_NROWS, _HIDDEN), jnp.float32),
        pltpu.SemaphoreType.DMA,
    ],
)
def _sc_gather(idx_hbm, table_hbm, out_hbm, idx_v, table_sh, sem):
    wid = lax.axis_index("s") * _NC + lax.axis_index("c")
    base = wid * _BPW

    @pl.when(lax.axis_index("s") == 0)
    def _():
        pltpu.sync_copy(table_hbm, table_sh)

    pltpu.sync_copy(idx_hbm.at[pl.ds(base, _BPW)], idx_v)
    plsc.subcore_barrier()

    def chunk(j, carry):
        for h in range(_CH // 16):
            vec = idx_v[pl.ds(j * _CH + h * 16, 16)]
            for i in range(16):
                pltpu.async_copy(
                    table_sh.at[vec[i]],
                    out_hbm.at[base + j * _CH + h * 16 + i],
                    sem)
        return carry

    lax.fori_loop(0, _NCHUNK, chunk, 0)
    pltpu.make_async_copy(
        out_hbm.at[pl.ds(base, _BPW)], out_hbm.at[pl.ds(base, _BPW)],
        sem).wait()


def kernel(ids, indicator_embs):
    ids_flat = ids.reshape(_B).astype(jnp.int32)
    out = _sc_gather(ids_flat, indicator_embs)
    return out.reshape(ids.shape + (_HIDDEN,))
